# Initial kernel scaffold; baseline (speedup 1.0000x reference)
import jax
import jax.numpy as jnp
from jax.experimental import pallas as pl


def kernel(prev, x, edge_index, W, att_src, att_dst, bias, gamma, beta):
    raise NotImplementedError("write your pallas kernel here")


# speedup vs baseline: 8.2828x; 8.2828x over previous
import dataclasses
import functools

import jax
import jax.numpy as jnp
from jax import lax
from jax.experimental import pallas as pl
from jax.experimental.pallas import tpu as pltpu
from jax.experimental.pallas import tpu_sc as plsc

N = 10000
E = 78000
D = 1024
H = 16
C = D // H
ETOT = E + N
EP = 90112
NP = 12800
RB = 400
NRB = N // RB
NB = 8
BW = 128
W_A = 2816
HALF = EP // 2
W_B = 128
NW_B = EP // 16 // W_B
STRIPE = NP // 16
NTAIL = STRIPE - (STRIPE // W_B) * W_B

_SC_PARAMS = pltpu.CompilerParams()
if "needs_layout_passes" in pltpu.CompilerParams.__dataclass_fields__:
    _SC_PARAMS = dataclasses.replace(_SC_PARAMS, needs_layout_passes=False)


def _stage1_body(x_ref, w_ref, a2_ref, xh_ref, a_ref, k_ref, kacc):
    i = pl.program_id(0)
    xh = jnp.dot(x_ref[...], w_ref[...], preferred_element_type=jnp.float32)
    for j in range(NB):
        xh_ref[j] = xh[:, j * BW:(j + 1) * BW]
    a = jnp.dot(xh, a2_ref[...], preferred_element_type=jnp.float32)
    a_ref[...] = a
    amax = jnp.max(a, axis=0, keepdims=True)

    @pl.when(i == 0)
    def _():
        kacc[...] = amax

    @pl.when(i > 0)
    def _():
        kacc[...] = jnp.maximum(kacc[...], amax)

    @pl.when(i == NRB - 1)
    def _():
        k_ref[...] = kacc[...]


def _stage1(x, w, a2):
    return pl.pallas_call(
        _stage1_body,
        grid=(NRB,),
        in_specs=[
            pl.BlockSpec((RB, D), lambda i: (i, 0)),
            pl.BlockSpec((D, D), lambda i: (0, 0)),
            pl.BlockSpec((D, 2 * H), lambda i: (0, 0)),
        ],
        out_specs=[
            pl.BlockSpec((NB, RB, BW), lambda i: (0, i, 0)),
            pl.BlockSpec((RB, 2 * H), lambda i: (i, 0)),
            pl.BlockSpec((1, 2 * H), lambda i: (0, 0)),
        ],
        out_shape=[
            jax.ShapeDtypeStruct((NB, N, BW), jnp.float32),
            jax.ShapeDtypeStruct((N, 2 * H), jnp.float32),
            jax.ShapeDtypeStruct((1, 2 * H), jnp.float32),
        ],
        scratch_shapes=[pltpu.VMEM((1, 2 * H), jnp.float32)],
    )(x, w, a2)


def _pass_a(asrc_t, adst_t, kb, src, dst):
    mesh = plsc.VectorSubcoreMesh(core_axis_name="c", subcore_axis_name="s")

    @functools.partial(
        pl.kernel,
        mesh=mesh,
        out_type=[
            jax.ShapeDtypeStruct((H * EP,), jnp.float32),
            jax.ShapeDtypeStruct((2 * H * NP,), jnp.float32),
        ],
        scratch_types=[
            pltpu.VMEM((NP,), jnp.float32),
            pltpu.VMEM((NP,), jnp.float32),
            pltpu.VMEM((NP,), jnp.float32),
            pltpu.VMEM((16,), jnp.float32),
            pltpu.VMEM((W_A,), jnp.int32),
            pltpu.VMEM((W_A,), jnp.int32),
            pltpu.VMEM((W_A,), jnp.float32),
        ],
        compiler_params=_SC_PARAMS,
    )
    def kfn(asrc_hbm, adst_hbm, k_hbm, src_hbm, dst_hbm, ea_hbm, den_hbm,
            at_ref, bt_ref, den_ref, k_ref, sw_ref, dw_ref, ea_ref):
        c = lax.axis_index("c")
        h = lax.axis_index("s")
        pltpu.sync_copy(asrc_hbm.at[pl.ds(h * NP, NP)], at_ref)
        pltpu.sync_copy(adst_hbm.at[pl.ds(h * NP, NP)], bt_ref)
        pltpu.sync_copy(k_hbm.at[pl.ds(h * 16, 16)], k_ref)
        kvec = k_ref[...]

        @pl.loop(0, NP, step=16)
        def _(i):
            den_ref[pl.ds(i, 16)] = jnp.zeros((16,), jnp.float32)

        base = c * HALF

        @pl.loop(0, HALF, step=W_A)
        def _(off):
            pltpu.sync_copy(src_hbm.at[pl.ds(base + off, W_A)], sw_ref)
            pltpu.sync_copy(dst_hbm.at[pl.ds(base + off, W_A)], dw_ref)

            @pl.loop(0, W_A, step=16)
            def _(k):
                sv = sw_ref[pl.ds(k, 16)]
                dv = dw_ref[pl.ds(k, 16)]
                asv = plsc.load_gather(at_ref, [sv])
                adv = plsc.load_gather(bt_ref, [dv])
                al = asv + adv
                al = jnp.where(al >= 0.0, al, 0.2 * al)
                ea = jnp.exp(al - kvec)
                ea_ref[pl.ds(k, 16)] = ea
                plsc.addupdate_scatter(den_ref, [dv], ea)

            pltpu.sync_copy(ea_ref, ea_hbm.at[pl.ds(h * EP + base + off, W_A)])

        pltpu.sync_copy(den_ref, den_hbm.at[pl.ds((c * H + h) * NP, NP)])

    return kfn(asrc_t, adst_t, kb, src, dst)


def _pass_b(xh8, src, dst, ea, den):
    mesh = plsc.VectorSubcoreMesh(core_axis_name="c", subcore_axis_name="s")

    @functools.partial(
        pl.kernel,
        mesh=mesh,
        out_type=jax.ShapeDtypeStruct((NB, NP, BW), jnp.float32),
        scratch_types=[
            pltpu.VMEM((W_B,), jnp.int32),
            pltpu.VMEM((W_B,), jnp.int32),
            pltpu.VMEM((W_B,), jnp.float32),
            pltpu.VMEM((W_B,), jnp.float32),
            pltpu.VMEM((W_B,), jnp.float32),
            pltpu.VMEM((W_B,), jnp.float32),
            pltpu.VMEM((W_B,), jnp.float32),
            pltpu.VMEM((W_B,), jnp.float32),
            pltpu.VMEM((W_B, BW), jnp.float32),
            pltpu.VMEM((NTAIL, BW), jnp.float32),
            pltpu.VMEM((STRIPE,), jnp.float32),
            pltpu.VMEM((STRIPE,), jnp.float32),
            pltpu.VMEM_SHARED((NP, BW), jnp.float32),
            pltpu.VMEM_SHARED((NP,), jnp.float32),
            pltpu.VMEM_SHARED((NP,), jnp.float32),
            pltpu.SemaphoreType.DMA,
        ],
        compiler_params=_SC_PARAMS,
    )
    def kfn(xh_hbm, src_hbm, dst_hbm, ea_hbm, den_hbm, out_hbm,
            sw, dw, e0, e1, d0, d1, c0, c1, gbuf, tailbuf, ca, cb, acc,
            den0_sh, den1_sh, sem):
        c = lax.axis_index("c")
        s = lax.axis_index("s")
        r0 = s * STRIPE
        ebase0 = s * (EP // 16)
        for jl in range(NB // 2):
            j = 2 * jl + c
            for hh, dsh in ((0, den0_sh), (1, den1_sh)):
                pltpu.sync_copy(
                    den_hbm.at[pl.ds((2 * j + hh) * NP + r0, STRIPE)], ca)
                pltpu.sync_copy(
                    den_hbm.at[pl.ds((H + 2 * j + hh) * NP + r0, STRIPE)], cb)

                @pl.loop(0, STRIPE, step=16)
                def _(i):
                    ca[pl.ds(i, 16)] = ca[pl.ds(i, 16)] + cb[pl.ds(i, 16)]

                pltpu.sync_copy(ca, dsh.at[pl.ds(r0, STRIPE)])

            @pl.loop(0, W_B)
            def _(r):
                grow = gbuf.at[r]
                for q in range(BW // 16):
                    grow[pl.ds(q * 16, 16)] = jnp.zeros((16,), jnp.float32)

            @pl.loop(0, STRIPE - NTAIL, step=W_B)
            def _(b):
                pltpu.sync_copy(gbuf, acc.at[pl.ds(r0 + b, W_B)])

            pltpu.sync_copy(gbuf.at[pl.ds(0, NTAIL)],
                            acc.at[pl.ds(r0 + STRIPE - NTAIL, NTAIL)])
            plsc.subcore_barrier()

            @pl.loop(0, NW_B)
            def _(w):
                eb = ebase0 + w * W_B
                pltpu.sync_copy(src_hbm.at[pl.ds(eb, W_B)], sw)
                pltpu.sync_copy(dst_hbm.at[pl.ds(eb, W_B)], dw)
                pltpu.sync_copy(ea_hbm.at[pl.ds(2 * j * EP + eb, W_B)], e0)
                pltpu.sync_copy(ea_hbm.at[pl.ds((2 * j + 1) * EP + eb, W_B)],
                                e1)
                pltpu.async_copy(xh_hbm.at[j].at[sw], gbuf, sem).wait()
                pltpu.async_copy(den0_sh.at[dw], d0, sem).wait()
                pltpu.async_copy(den1_sh.at[dw], d1, sem).wait()

                @pl.loop(0, W_B, step=16)
                def _(g):
                    sl = pl.ds(g, 16)
                    c0[sl] = e0[sl] / (d0[sl] + 1e-16)
                    c1[sl] = e1[sl] / (d1[sl] + 1e-16)

                @pl.loop(0, W_B)
                def _(e):
                    b0 = plsc.load_gather(c0, [jnp.full((16,), e, jnp.int32)])
                    b1 = plsc.load_gather(c1, [jnp.full((16,), e, jnp.int32)])
                    row = gbuf.at[e]
                    for q in range(4):
                        row[pl.ds(q * 16, 16)] = row[pl.ds(q * 16, 16)] * b0
                    for q in range(4, 8):
                        row[pl.ds(q * 16, 16)] = row[pl.ds(q * 16, 16)] * b1

                pltpu.sync_copy(gbuf, acc.at[dw], add=True)

            plsc.subcore_barrier()

            @pl.loop(0, STRIPE - NTAIL, step=W_B)
            def _(b):
                pltpu.sync_copy(acc.at[pl.ds(r0 + b, W_B)], gbuf)
                pltpu.sync_copy(gbuf, out_hbm.at[j, pl.ds(r0 + b, W_B)])

            pltpu.sync_copy(acc.at[pl.ds(r0 + STRIPE - NTAIL, NTAIL)], tailbuf)
            pltpu.sync_copy(tailbuf, out_hbm.at[j, pl.ds(r0 + STRIPE - NTAIL,
                                                         NTAIL)])
            plsc.subcore_barrier()

    return kfn(xh8, src, dst, ea, den)


def _stage3a_body(prev_ref, out_ref, bias_ref, s_ref, ss_ref, sacc, ssacc):
    i = pl.program_id(0)
    s_parts = []
    ss_parts = []
    for jj in range(NB):
        y = (prev_ref[:, jj * BW:(jj + 1) * BW] + out_ref[jj]
             + bias_ref[jj][None, :])
        s_parts.append(jnp.sum(y, axis=0, keepdims=True))
        ss_parts.append(jnp.sum(y * y, axis=0, keepdims=True))
    ssum = jnp.concatenate(s_parts, axis=0)
    sssum = jnp.concatenate(ss_parts, axis=0)

    @pl.when(i == 0)
    def _():
        sacc[...] = ssum
        ssacc[...] = sssum

    @pl.when(i > 0)
    def _():
        sacc[...] = sacc[...] + ssum
        ssacc[...] = ssacc[...] + sssum

    @pl.when(i == NRB - 1)
    def _():
        s_ref[...] = sacc[...]
        ss_ref[...] = ssacc[...]


def _stage3a(prev, out, bias8):
    return pl.pallas_call(
        _stage3a_body,
        grid=(NRB,),
        in_specs=[
            pl.BlockSpec((RB, D), lambda i: (i, 0)),
            pl.BlockSpec((NB, RB, BW), lambda i: (0, i, 0)),
            pl.BlockSpec((NB, BW), lambda i: (0, 0)),
        ],
        out_specs=[
            pl.BlockSpec((NB, BW), lambda i: (0, 0)),
            pl.BlockSpec((NB, BW), lambda i: (0, 0)),
        ],
        out_shape=[
            jax.ShapeDtypeStruct((NB, BW), jnp.float32),
            jax.ShapeDtypeStruct((NB, BW), jnp.float32),
        ],
        scratch_shapes=[
            pltpu.VMEM((NB, BW), jnp.float32),
            pltpu.VMEM((NB, BW), jnp.float32),
        ],
    )(prev, out, bias8)


def _stage3b_body(prev_ref, out_ref, bias_ref, g_ref, bt_ref, s_ref, ss_ref,
                  y_ref):
    j = pl.program_id(0)
    row = lambda ref: ref[pl.ds(j, 1), :]
    inv_n = 1.0 / N
    mean = row(s_ref) * inv_n
    var = row(ss_ref) * inv_n - mean * mean
    scale = row(g_ref) * lax.rsqrt(var + 1e-5)
    shift = row(bt_ref) - mean * scale
    y = prev_ref[...] + out_ref[0] + row(bias_ref)
    y = y * scale + shift
    y_ref[...] = jnp.maximum(y, 0.0)


def _stage3b(prev, out, bias8, gamma8, beta8, sums, sumsq):
    return pl.pallas_call(
        _stage3b_body,
        grid=(NB, NRB),
        in_specs=[
            pl.BlockSpec((RB, BW), lambda j, i: (i, j)),
            pl.BlockSpec((1, RB, BW), lambda j, i: (j, i, 0)),
            pl.BlockSpec((NB, BW), lambda j, i: (0, 0)),
            pl.BlockSpec((NB, BW), lambda j, i: (0, 0)),
            pl.BlockSpec((NB, BW), lambda j, i: (0, 0)),
            pl.BlockSpec((NB, BW), lambda j, i: (0, 0)),
            pl.BlockSpec((NB, BW), lambda j, i: (0, 0)),
        ],
        out_specs=pl.BlockSpec((RB, BW), lambda j, i: (i, j)),
        out_shape=jax.ShapeDtypeStruct((N, D), jnp.float32),
    )(prev, out, bias8, gamma8, beta8, sums, sumsq)


def kernel(prev, x, edge_index, W, att_src, att_dst, bias, gamma, beta):
    loop = jnp.arange(N, dtype=edge_index.dtype)
    ei = jnp.concatenate([edge_index, jnp.stack([loop, loop])], axis=1)
    src = ei[0].astype(jnp.int32)
    dst = ei[1].astype(jnp.int32)
    npad = EP - ETOT
    pad_src = jnp.zeros((npad,), jnp.int32)
    pad_dst = N + (jnp.arange(npad, dtype=jnp.int32) % (NP - N))
    src = jnp.concatenate([src, pad_src])
    dst = jnp.concatenate([dst, pad_dst])

    att_s = att_src.reshape(H, C)
    att_d = att_dst.reshape(H, C)
    eye = jnp.eye(H, dtype=jnp.float32)
    a2 = jnp.concatenate(
        [
            (att_s[:, :, None] * eye[:, None, :]).reshape(D, H),
            (att_d[:, :, None] * eye[:, None, :]).reshape(D, H),
        ],
        axis=1,
    )

    xh8, a, kmax = _stage1(x, W, a2)

    a_t = a.T
    asrc_t = jnp.zeros((H, NP), jnp.float32).at[:, :N].set(a_t[:H]).reshape(-1)
    adst_t = jnp.zeros((H, NP), jnp.float32).at[:, :N].set(a_t[H:]).reshape(-1)
    ksum = kmax[0, :H] + kmax[0, H:]
    shift = jnp.where(ksum >= 0.0, ksum, 0.2 * ksum)
    kb = jnp.broadcast_to(shift[:, None], (H, 16)).reshape(-1)

    ea, den = _pass_a(asrc_t, adst_t, kb, src, dst)
    out8 = _pass_b(xh8, src, dst, ea, den)

    bias8 = bias.reshape(NB, BW)
    gamma8 = gamma.reshape(NB, BW)
    beta8 = beta.reshape(NB, BW)
    sums, sumsq = _stage3a(prev, out8, bias8)
    return _stage3b(prev, out8, bias8, gamma8, beta8, sums, sumsq)
